# baseline (speedup 1.0000x reference)
import functools

import jax
import jax.numpy as jnp
from jax import lax
from jax.experimental import pallas as pl
from jax.experimental.pallas import tpu as pltpu
from jax.experimental.pallas import tpu_sc as plsc

B, A, D, F_ = 16, 2048, 256, 128
TEMP = 0.1
EPS = 1e-8

SPLIT = 8
BT = B - SPLIT
NW = 32
WPI = NW // SPLIT
ROWS_W = (SPLIT * A) // NW
CHUNK = 128
NCHUNK = ROWS_W // CHUNK
LANES = 16

NSTREAM = 4
GROUP = CHUNK // NSTREAM


def _seg_body(ae_hbm, idx_hbm, out_hbm, cnt_hbm, idx_v, buf, acc, acc_cnt,
              sems):
    c = lax.axis_index("c")
    s = lax.axis_index("s")
    wid = c * 16 + s
    base = wid * ROWS_W

    pltpu.async_copy(ae_hbm.at[pl.ds(base, CHUNK)], buf.at[0], sems[0])
    pltpu.async_copy(ae_hbm.at[pl.ds(base + CHUNK, CHUNK)], buf.at[1], sems[1])
    pltpu.sync_copy(idx_hbm.at[pl.ds(base, ROWS_W)], idx_v)

    zero = jnp.zeros((LANES,), jnp.float32)

    @plsc.parallel_loop(0, F_, 1, unroll=4)
    def _(r):
        for j in range(D // LANES):
            acc[r, pl.ds(j * LANES, LANES)] = zero

    for j in range(F_ // LANES):
        acc_cnt[0, pl.ds(j * LANES, LANES)] = zero

    cols = [lax.iota(jnp.int32, 16) + j * LANES for j in range(D // LANES)]
    lane0 = lax.iota(jnp.int32, 16) == 0
    row0 = jnp.zeros((LANES,), jnp.int32)
    ones = jnp.ones((LANES,), jnp.float32)

    def outer(kk, _):
        for slot in range(2):
            chunk_id = kk * 2 + slot
            chunk_base = chunk_id * CHUNK
            pltpu.make_async_copy(
                ae_hbm.at[pl.ds(base, CHUNK)], buf.at[slot],
                sems[slot]).wait()
            bufk = buf.at[slot]

            @plsc.parallel_loop(0, GROUP, 1, unroll=1)
            def _(g, bufk=bufk, chunk_base=chunk_base):
                for h in range(NSTREAM):
                    rowv = plsc.load_gather(
                        idx_v,
                        [jnp.full((LANES,), chunk_base + g + h * GROUP,
                                  jnp.int32)])
                    plsc.addupdate_scatter(acc_cnt, [row0, rowv], ones,
                                           mask=lane0)
                    for j in range(D // LANES):
                        x = bufk[g + h * GROUP, pl.ds(j * LANES, LANES)]
                        plsc.addupdate_scatter(acc, [rowv, cols[j]], x)

            @pl.when(chunk_id + 2 < NCHUNK)
            def _(chunk_base=chunk_base, slot=slot):
                pltpu.async_copy(
                    ae_hbm.at[pl.ds(base + chunk_base + 2 * CHUNK, CHUNK)],
                    buf.at[slot], sems[slot])
        return 0

    lax.fori_loop(0, NCHUNK // 2, outer, 0)

    pltpu.sync_copy(acc, out_hbm.at[pl.ds(wid * F_, F_)])
    pltpu.sync_copy(acc_cnt.at[0], cnt_hbm.at[pl.ds(wid * F_, F_)])


def _segment_sums_sc(atom_embed, index):
    mesh = plsc.VectorSubcoreMesh(core_axis_name="c", subcore_axis_name="s")
    k = pl.kernel(
        _seg_body,
        out_type=(jax.ShapeDtypeStruct((NW * F_, D), jnp.float32),
                  jax.ShapeDtypeStruct((NW * F_,), jnp.float32)),
        mesh=mesh,
        compiler_params=pltpu.CompilerParams(needs_layout_passes=False),
        scratch_types=[
            pltpu.VMEM((ROWS_W,), jnp.int32),
            pltpu.VMEM((2, CHUNK, D), jnp.float32),
            pltpu.VMEM((F_, D), jnp.float32),
            pltpu.VMEM((8, F_), jnp.float32),
            (pltpu.SemaphoreType.DMA, pltpu.SemaphoreType.DMA),
        ],
    )
    return k(atom_embed[:SPLIT].reshape(SPLIT * A, D),
             index[:SPLIT].reshape(SPLIT * A))


def _tcseg_body(idx_ref, ae_ref, sums_ref, cnt_ref):
    idx = idx_ref[0, 0]
    ae = ae_ref[0]
    frag_ids = lax.broadcasted_iota(jnp.int32, (A, F_), 1)
    onehot = (idx[:, None] == frag_ids).astype(jnp.float32)
    sums_ref[0] = lax.dot_general(onehot, ae, (((0,), (0,)), ((), ())),
                                  preferred_element_type=jnp.float32)
    cnt_ref[0, 0] = jnp.sum(onehot, axis=0)


def _segment_sums_tc(atom_embed, index):
    return pl.pallas_call(
        _tcseg_body,
        grid=(BT,),
        in_specs=[
            pl.BlockSpec((1, 1, A), lambda b: (b, 0, 0)),
            pl.BlockSpec((1, A, D), lambda b: (b, 0, 0)),
        ],
        out_specs=[
            pl.BlockSpec((1, F_, D), lambda b: (b, 0, 0)),
            pl.BlockSpec((1, 1, F_), lambda b: (b, 0, 0)),
        ],
        out_shape=[
            jax.ShapeDtypeStruct((BT, F_, D), jnp.float32),
            jax.ShapeDtypeStruct((BT, 1, F_), jnp.float32),
        ],
    )(index[SPLIT:].reshape(BT, 1, A), atom_embed[SPLIT:])


def _item_loss(sums, counts, fe, eye):
    valid = counts > 0.0
    mean = sums / jnp.maximum(counts, 1.0)[:, None]
    mn = jnp.maximum(jnp.sqrt(jnp.sum(mean * mean, axis=1,
                                      keepdims=True)), EPS)
    fn = jnp.maximum(jnp.sqrt(jnp.sum(fe * fe, axis=1,
                                      keepdims=True)), EPS)
    sims = lax.dot_general(mean / mn, fe / fn, (((1,), (1,)), ((), ())),
                           preferred_element_type=jnp.float32) / TEMP

    pos = jnp.sum(sims * eye, axis=1)
    lse = jnp.log(jnp.sum(jnp.exp(sims), axis=1))
    loss_f = lse - pos

    return (jnp.sum(jnp.where(valid, loss_f, 0.0)),
            jnp.sum(valid.astype(jnp.float32)))


def _dense_body(cparts_ref, parts_ref, sums_tc_ref, cnt_tc_ref, fe_ref,
                loss_ref, cnt_ref):
    eye = (lax.broadcasted_iota(jnp.int32, (F_, F_), 0)
           == lax.broadcasted_iota(jnp.int32, (F_, F_), 1)).astype(jnp.float32)

    total_loss = jnp.float32(0.0)
    total_cnt = jnp.float32(0.0)
    for i in range(SPLIT):
        sums = parts_ref[WPI * i]
        counts = cparts_ref[WPI * i]
        for w in range(1, WPI):
            sums = sums + parts_ref[WPI * i + w]
            counts = counts + cparts_ref[WPI * i + w]
        l, c = _item_loss(sums, counts, fe_ref[i], eye)
        total_loss += l
        total_cnt += c
    for i in range(BT):
        l, c = _item_loss(sums_tc_ref[i], cnt_tc_ref[i, 0],
                          fe_ref[SPLIT + i], eye)
        total_loss += l
        total_cnt += c

    loss_ref[...] = total_loss.reshape(1, 1)
    cnt_ref[...] = total_cnt.reshape(1, 1)


def kernel(atom_embed, fragment_embed, index):
    parts, cparts = _segment_sums_sc(atom_embed, index)
    sums_tc, cnt_tc = _segment_sums_tc(atom_embed, index)
    loss, cnt = pl.pallas_call(
        _dense_body,
        grid=(1,),
        in_specs=[
            pl.BlockSpec((NW, F_), lambda b: (0, 0)),
            pl.BlockSpec((NW, F_, D), lambda b: (0, 0, 0)),
            pl.BlockSpec((BT, F_, D), lambda b: (0, 0, 0)),
            pl.BlockSpec((BT, 1, F_), lambda b: (0, 0, 0)),
            pl.BlockSpec((B, F_, D), lambda b: (0, 0, 0)),
        ],
        out_specs=[
            pl.BlockSpec((1, 1), lambda b: (0, 0)),
            pl.BlockSpec((1, 1), lambda b: (0, 0)),
        ],
        out_shape=[
            jax.ShapeDtypeStruct((1, 1), jnp.float32),
            jax.ShapeDtypeStruct((1, 1), jnp.float32),
        ],
    )(cparts.reshape(NW, F_), parts.reshape(NW, F_, D), sums_tc, cnt_tc,
      fragment_embed)
    total = loss[0, 0]
    c = cnt[0, 0]
    return jnp.where(c > 0, total / c, jnp.float32(0.0))


# speedup vs baseline: 1.2514x; 1.2514x over previous
import jax
import jax.numpy as jnp
from jax import lax
from jax.experimental import pallas as pl
from jax.experimental.pallas import tpu as pltpu
from jax.experimental.pallas import tpu_sc as plsc

B, A, D, F_ = 16, 2048, 256, 128
TEMP = 0.1
EPS = 1e-8

NW = 32
ROWS_W = (B * A) // NW
CHUNK = 128
NCHUNK = ROWS_W // CHUNK
LANES = 16


NSTREAM = 4
GROUP = CHUNK // NSTREAM


def _seg_body(ae_hbm, idx_hbm, out_hbm, cnt_hbm, idx_v, buf, acc, acc_cnt,
              sems):
    c = lax.axis_index("c")
    s = lax.axis_index("s")
    wid = c * 16 + s
    base = wid * ROWS_W

    pltpu.async_copy(ae_hbm.at[pl.ds(base, CHUNK)], buf.at[0], sems[0])
    pltpu.async_copy(ae_hbm.at[pl.ds(base + CHUNK, CHUNK)], buf.at[1], sems[1])
    pltpu.sync_copy(idx_hbm.at[pl.ds(base, ROWS_W)], idx_v)

    zero = jnp.zeros((LANES,), jnp.float32)

    @plsc.parallel_loop(0, F_, 1, unroll=4)
    def _(r):
        for j in range(D // LANES):
            acc[r, pl.ds(j * LANES, LANES)] = zero

    for j in range(F_ // LANES):
        acc_cnt[0, pl.ds(j * LANES, LANES)] = zero

    cols = [lax.iota(jnp.int32, 16) + j * LANES for j in range(D // LANES)]
    lane0 = lax.iota(jnp.int32, 16) == 0
    row0 = jnp.zeros((LANES,), jnp.int32)
    ones = jnp.ones((LANES,), jnp.float32)

    def outer(kk, _):
        for slot in range(2):
            chunk_id = kk * 2 + slot
            chunk_base = chunk_id * CHUNK
            pltpu.make_async_copy(
                ae_hbm.at[pl.ds(base, CHUNK)], buf.at[slot],
                sems[slot]).wait()
            bufk = buf.at[slot]

            @plsc.parallel_loop(0, GROUP, 1, unroll=1)
            def _(g, bufk=bufk, chunk_base=chunk_base):
                for h in range(NSTREAM):
                    rowv = plsc.load_gather(
                        idx_v,
                        [jnp.full((LANES,), chunk_base + g + h * GROUP,
                                  jnp.int32)])
                    plsc.addupdate_scatter(acc_cnt, [row0, rowv], ones,
                                           mask=lane0)
                    for j in range(D // LANES):
                        x = bufk[g + h * GROUP, pl.ds(j * LANES, LANES)]
                        plsc.addupdate_scatter(acc, [rowv, cols[j]], x)

            @pl.when(chunk_id + 2 < NCHUNK)
            def _(chunk_base=chunk_base, slot=slot):
                pltpu.async_copy(
                    ae_hbm.at[pl.ds(base + chunk_base + 2 * CHUNK, CHUNK)],
                    buf.at[slot], sems[slot])
        return 0

    lax.fori_loop(0, NCHUNK // 2, outer, 0)

    pltpu.sync_copy(acc, out_hbm.at[pl.ds(wid * F_, F_)])
    pltpu.sync_copy(acc_cnt.at[0], cnt_hbm.at[pl.ds(wid * F_, F_)])


def _segment_sums(atom_embed, index):
    mesh = plsc.VectorSubcoreMesh(core_axis_name="c", subcore_axis_name="s")
    k = pl.kernel(
        _seg_body,
        out_type=(jax.ShapeDtypeStruct((NW * F_, D), jnp.float32),
                  jax.ShapeDtypeStruct((NW * F_,), jnp.float32)),
        mesh=mesh,
        compiler_params=pltpu.CompilerParams(needs_layout_passes=False),
        scratch_types=[
            pltpu.VMEM((ROWS_W,), jnp.int32),
            pltpu.VMEM((2, CHUNK, D), jnp.float32),
            pltpu.VMEM((F_, D), jnp.float32),
            pltpu.VMEM((8, F_), jnp.float32),
            (pltpu.SemaphoreType.DMA, pltpu.SemaphoreType.DMA),
        ],
    )
    return k(atom_embed.reshape(B * A, D), index.reshape(B * A))


IB = 8


def _dense_body(cparts_ref, parts_ref, fe_ref, loss_ref, cnt_ref):
    b = pl.program_id(0)

    eye = (lax.broadcasted_iota(jnp.int32, (F_, F_), 0)
           == lax.broadcasted_iota(jnp.int32, (F_, F_), 1)).astype(jnp.float32)

    item_loss = jnp.float32(0.0)
    item_cnt = jnp.float32(0.0)
    for i in range(IB):
        sums = parts_ref[2 * i] + parts_ref[2 * i + 1]
        fe = fe_ref[i]
        counts = cparts_ref[2 * i] + cparts_ref[2 * i + 1]

        valid = counts > 0.0
        mean = sums / jnp.maximum(counts, 1.0)[:, None]
        mn = jnp.maximum(jnp.sqrt(jnp.sum(mean * mean, axis=1,
                                          keepdims=True)), EPS)
        fn = jnp.maximum(jnp.sqrt(jnp.sum(fe * fe, axis=1,
                                          keepdims=True)), EPS)
        sims = lax.dot_general(mean / mn, fe / fn, (((1,), (1,)), ((), ())),
                               preferred_element_type=jnp.float32) / TEMP

        pos = jnp.sum(sims * eye, axis=1)
        lse = jnp.log(jnp.sum(jnp.exp(sims), axis=1))
        loss_f = lse - pos

        item_loss += jnp.sum(jnp.where(valid, loss_f, 0.0))
        item_cnt += jnp.sum(valid.astype(jnp.float32))

    @pl.when(b == 0)
    def _():
        loss_ref[...] = jnp.zeros_like(loss_ref)
        cnt_ref[...] = jnp.zeros_like(cnt_ref)

    loss_ref[...] += item_loss.reshape(1, 1)
    cnt_ref[...] += item_cnt.reshape(1, 1)


def kernel(atom_embed, fragment_embed, index):
    parts, cparts = _segment_sums(atom_embed, index)
    parts = parts.reshape(NW, F_, D)
    cparts = cparts.reshape(NW, F_)
    loss, cnt = pl.pallas_call(
        _dense_body,
        grid=(B // IB,),
        in_specs=[
            pl.BlockSpec((2 * IB, F_), lambda b: (b, 0)),
            pl.BlockSpec((2 * IB, F_, D), lambda b: (b, 0, 0)),
            pl.BlockSpec((IB, F_, D), lambda b: (b, 0, 0)),
        ],
        out_specs=[
            pl.BlockSpec((1, 1), lambda b: (0, 0)),
            pl.BlockSpec((1, 1), lambda b: (0, 0)),
        ],
        out_shape=[
            jax.ShapeDtypeStruct((1, 1), jnp.float32),
            jax.ShapeDtypeStruct((1, 1), jnp.float32),
        ],
    )(cparts, parts, fragment_embed)
    total = loss[0, 0]
    c = cnt[0, 0]
    return jnp.where(c > 0, total / c, jnp.float32(0.0))
